# baseline (speedup 1.0000x reference)
import math

import jax
import jax.numpy as jnp
from jax.experimental import pallas as pl
from jax.experimental.pallas import tpu as pltpu

_N = 2048
_F = 256
_L = 8
_ALPHA = 0.1
_THETA = 0.5
_RBLK = 256
_CHUNK = 512
_STEPS = _N // _RBLK

_BETAS = [math.log(_THETA / (l + 1) + 1.0) for l in range(_L)]
_G_LAST = 1.0 - _BETAS[_L - 1]


def _gcnii_body(x_ref, adj_ref, W0_ref, b0_ref, W1_ref, b1_ref, Wc_ref,
                out_ref, A_ref, usa_ref, usb_ref, h0a_ref, ht_ref,
                mt_ref, mb_ref, deg_ref, Wlp_ref, W1g_ref):
    f32 = jnp.float32
    bf16 = jnp.bfloat16
    step = pl.program_id(0)

    @pl.when(step < _STEPS)
    def _build():
        a = adj_ref[...].astype(f32)
        col = jax.lax.broadcasted_iota(jnp.int32, (_RBLK, _N), 1)
        row = jax.lax.broadcasted_iota(jnp.int32, (_RBLK, _N), 0)
        a = jnp.where(col == row + step * _RBLK, 1.0, a)
        prev = jnp.where(step == 0, jnp.zeros((1, _N), f32), deg_ref[...])
        deg_ref[...] = prev + jnp.sum(a, axis=0, keepdims=True)
        A_ref[pl.ds(step * _RBLK, _RBLK), :] = a.astype(bf16)

        for l in range(_L):
            g = 1.0 - _BETAS[l]

            @pl.when(step == l)
            def _wprep(l=l, g=g):
                Wlp_ref[l] = ((_BETAS[l] / g) *
                              jnp.transpose(Wc_ref[l])).astype(bf16)

    @pl.when(step == 0)
    def _input_layer():
        xt = jnp.transpose(x_ref[...])
        b0c = jnp.transpose(b0_ref[...])
        h0 = jnp.dot(W0_ref[...].astype(bf16), xt.astype(bf16),
                     preferred_element_type=f32) + b0c
        h0 = jnp.maximum(h0, 0.0)
        ht_ref[...] = h0
        h0a_ref[...] = _ALPHA * h0

    @pl.when(step == 1)
    def _w1prep():
        W1g_ref[...] = (_G_LAST * jnp.transpose(W1_ref[...])).astype(bf16)

    @pl.when(step == _STEPS)
    def _layers():
        dinv = jax.lax.rsqrt(deg_ref[...])
        dv09 = (1.0 - _ALPHA) * dinv

        usa_ref[...] = (dinv * ht_ref[...]).astype(bf16)

        us_refs = [usa_ref, usb_ref]
        for l in range(_L):
            g = 1.0 - _BETAS[l]
            src = us_refs[l % 2]
            dst = us_refs[(l + 1) % 2]
            Wl = Wlp_ref[l]
            dvg = g * dinv
            us = src[...]
            for j in range(_N // _CHUNK):
                c0 = j * _CHUNK
                mm = jnp.dot(us, A_ref[:, c0:c0 + _CHUNK],
                             preferred_element_type=f32)
                m = mm * dv09[:, c0:c0 + _CHUNK] + h0a_ref[:, c0:c0 + _CHUNK]
                mt_ref[:, c0:c0 + _CHUNK] = m
                mb_ref[:, c0:c0 + _CHUNK] = m.astype(bf16)
            for j in range(_N // _CHUNK):
                c0 = j * _CHUNK
                s = jnp.dot(Wl, mb_ref[:, c0:c0 + _CHUNK],
                            preferred_element_type=f32)
                hs = jnp.maximum(mt_ref[:, c0:c0 + _CHUNK] + s, 0.0)
                if l < _L - 1:
                    dst[:, c0:c0 + _CHUNK] = (hs * dvg[:, c0:c0 + _CHUNK]
                                              ).astype(bf16)
                else:
                    ht_ref[:, c0:c0 + _CHUNK] = hs

        htf = jnp.transpose(ht_ref[...])
        out_ref[...] = jnp.dot(htf.astype(bf16), W1g_ref[...],
                               preferred_element_type=f32) + b1_ref[...]


def _run(x, adj, W0, b0r, W1, b1r, Wc, interpret=False):
    return pl.pallas_call(
        _gcnii_body,
        grid=(_STEPS + 1,),
        in_specs=[
            pl.BlockSpec((_N, _F), lambda i: (0, 0)),
            pl.BlockSpec((_RBLK, _N),
                         lambda i: (jnp.minimum(i, _STEPS - 1), 0)),
            pl.BlockSpec((_F, _F), lambda i: (0, 0)),
            pl.BlockSpec((1, _F), lambda i: (0, 0)),
            pl.BlockSpec((_F, _F), lambda i: (0, 0)),
            pl.BlockSpec((1, _F), lambda i: (0, 0)),
            pl.BlockSpec((_L, _F, _F), lambda i: (0, 0, 0)),
        ],
        out_specs=pl.BlockSpec((_N, _F), lambda i: (0, 0)),
        out_shape=jax.ShapeDtypeStruct((_N, _F), jnp.float32),
        scratch_shapes=[
            pltpu.VMEM((_N, _N), jnp.bfloat16),
            pltpu.VMEM((_F, _N), jnp.bfloat16),
            pltpu.VMEM((_F, _N), jnp.bfloat16),
            pltpu.VMEM((_F, _N), jnp.float32),
            pltpu.VMEM((_F, _N), jnp.float32),
            pltpu.VMEM((_F, _N), jnp.float32),
            pltpu.VMEM((_F, _N), jnp.bfloat16),
            pltpu.VMEM((1, _N), jnp.float32),
            pltpu.VMEM((_L, _F, _F), jnp.bfloat16),
            pltpu.VMEM((_F, _F), jnp.bfloat16),
        ],
        interpret=interpret,
    )(x, adj, W0, b0r, W1, b1r, Wc)


def kernel(x, adj, W0, b0, W1, b1, Wc):
    return _run(x, adj, W0, b0.reshape(1, _F), W1, b1.reshape(1, _F), Wc)


# speedup vs baseline: 1.0136x; 1.0136x over previous
import math

import jax
import jax.numpy as jnp
from jax.experimental import pallas as pl
from jax.experimental.pallas import tpu as pltpu

_N = 2048
_F = 256
_L = 8
_ALPHA = 0.1
_THETA = 0.5
_RBLK = 256
_CHUNK = 512
_STEPS = _N // _RBLK

_BETAS = [math.log(_THETA / (l + 1) + 1.0) for l in range(_L)]
_G_LAST = 1.0 - _BETAS[_L - 1]


def _gcnii_body(x_ref, adj_ref, W0_ref, b0_ref, W1_ref, b1_ref, Wc_ref,
                out_ref, A_ref, usa_ref, usb_ref, h0a_ref, ht_ref,
                mt_ref, mb_ref, deg_ref, W1g_ref):
    f32 = jnp.float32
    bf16 = jnp.bfloat16
    step = pl.program_id(0)

    @pl.when(step < _STEPS)
    def _build():
        a = adj_ref[...].astype(f32)
        col = jax.lax.broadcasted_iota(jnp.int32, (_RBLK, _N), 1)
        row = jax.lax.broadcasted_iota(jnp.int32, (_RBLK, _N), 0)
        a = jnp.where(col == row + step * _RBLK, 1.0, a)
        prev = jnp.where(step == 0, jnp.zeros((1, _N), f32), deg_ref[...])
        deg_ref[...] = prev + jnp.sum(a, axis=0, keepdims=True)
        A_ref[pl.ds(step * _RBLK, _RBLK), :] = a.astype(bf16)

    @pl.when(step == 0)
    def _input_layer():
        xt = jnp.transpose(x_ref[...])
        b0c = jnp.transpose(b0_ref[...])
        h0 = jnp.dot(W0_ref[...].astype(bf16), xt.astype(bf16),
                     preferred_element_type=f32) + b0c
        h0 = jnp.maximum(h0, 0.0)
        ht_ref[...] = h0
        h0a_ref[...] = _ALPHA * h0

    @pl.when(step == 1)
    def _w1prep():
        W1g_ref[...] = (_G_LAST * jnp.transpose(W1_ref[...])).astype(bf16)

    @pl.when(step == _STEPS)
    def _layers():
        dinv = jax.lax.rsqrt(deg_ref[...])
        dv09 = (1.0 - _ALPHA) * dinv

        usa_ref[...] = (dinv * ht_ref[...]).astype(bf16)

        us_refs = [usa_ref, usb_ref]
        for l in range(_L):
            g = 1.0 - _BETAS[l]
            src = us_refs[l % 2]
            dst = us_refs[(l + 1) % 2]
            Wl = ((_BETAS[l] / g) * jnp.transpose(Wc_ref[l])).astype(bf16)
            dvg = g * dinv
            us = src[...]
            for j in range(_N // _CHUNK):
                c0 = j * _CHUNK
                mm = jnp.dot(us, A_ref[:, c0:c0 + _CHUNK],
                             preferred_element_type=f32)
                m = mm * dv09[:, c0:c0 + _CHUNK] + h0a_ref[:, c0:c0 + _CHUNK]
                mt_ref[:, c0:c0 + _CHUNK] = m
                mb_ref[:, c0:c0 + _CHUNK] = m.astype(bf16)
            for j in range(_N // _CHUNK):
                c0 = j * _CHUNK
                s = jnp.dot(Wl, mb_ref[:, c0:c0 + _CHUNK],
                            preferred_element_type=f32)
                hs = jnp.maximum(mt_ref[:, c0:c0 + _CHUNK] + s, 0.0)
                if l < _L - 1:
                    dst[:, c0:c0 + _CHUNK] = (hs * dvg[:, c0:c0 + _CHUNK]
                                              ).astype(bf16)
                else:
                    ht_ref[:, c0:c0 + _CHUNK] = hs

        htf = jnp.transpose(ht_ref[...])
        out_ref[...] = jnp.dot(htf.astype(bf16), W1g_ref[...],
                               preferred_element_type=f32) + b1_ref[...]


def _run(x, adj, W0, b0r, W1, b1r, Wc, interpret=False):
    return pl.pallas_call(
        _gcnii_body,
        grid=(_STEPS + 1,),
        in_specs=[
            pl.BlockSpec((_N, _F), lambda i: (0, 0)),
            pl.BlockSpec((_RBLK, _N),
                         lambda i: (jnp.minimum(i, _STEPS - 1), 0)),
            pl.BlockSpec((_F, _F), lambda i: (0, 0)),
            pl.BlockSpec((1, _F), lambda i: (0, 0)),
            pl.BlockSpec((_F, _F), lambda i: (0, 0)),
            pl.BlockSpec((1, _F), lambda i: (0, 0)),
            pl.BlockSpec((_L, _F, _F), lambda i: (0, 0, 0)),
        ],
        out_specs=pl.BlockSpec((_N, _F), lambda i: (0, 0)),
        out_shape=jax.ShapeDtypeStruct((_N, _F), jnp.float32),
        scratch_shapes=[
            pltpu.VMEM((_N, _N), jnp.bfloat16),
            pltpu.VMEM((_F, _N), jnp.bfloat16),
            pltpu.VMEM((_F, _N), jnp.bfloat16),
            pltpu.VMEM((_F, _N), jnp.float32),
            pltpu.VMEM((_F, _N), jnp.float32),
            pltpu.VMEM((_F, _N), jnp.float32),
            pltpu.VMEM((_F, _N), jnp.bfloat16),
            pltpu.VMEM((1, _N), jnp.float32),
            pltpu.VMEM((_F, _F), jnp.bfloat16),
        ],
        interpret=interpret,
    )(x, adj, W0, b0r, W1, b1r, Wc)


def kernel(x, adj, W0, b0, W1, b1, Wc):
    return _run(x, adj, W0, b0.reshape(1, _F), W1, b1.reshape(1, _F), Wc)


# speedup vs baseline: 1.0407x; 1.0267x over previous
import math

import jax
import jax.numpy as jnp
from jax.experimental import pallas as pl
from jax.experimental.pallas import tpu as pltpu

_N = 2048
_F = 256
_L = 8
_ALPHA = 0.1
_THETA = 0.5
_RBLK = 256
_CHUNK = 512
_STEPS = _N // _RBLK


def _gcnii_body(x_ref, adj_ref, W0_ref, b0_ref, W1_ref, b1_ref, Wc_ref,
                out_ref, A_ref, usa_ref, usb_ref, h0a_ref, ht_ref,
                mt_ref, deg_ref):
    f32 = jnp.float32
    bf16 = jnp.bfloat16
    step = pl.program_id(0)

    @pl.when(step < _STEPS)
    def _build():
        a = adj_ref[...].astype(f32)
        prev = jnp.where(step == 0, jnp.zeros((1, _N), f32), deg_ref[...])
        deg_ref[...] = prev + jnp.sum(a, axis=0, keepdims=True)
        A_ref[pl.ds(step * _RBLK, _RBLK), :] = a.astype(bf16)

    @pl.when(step == 0)
    def _input_layer():
        xt = jnp.transpose(x_ref[...])
        b0c = jnp.transpose(b0_ref[...])
        h0 = jnp.dot(W0_ref[...].astype(bf16), xt.astype(bf16),
                     preferred_element_type=f32) + b0c
        h0 = jnp.maximum(h0, 0.0)
        ht_ref[...] = h0
        h0a_ref[...] = _ALPHA * h0

    @pl.when(step == _STEPS)
    def _layers():
        corrs = []
        for i in range(_STEPS):
            r0 = i * _RBLK
            dsub = A_ref[r0:r0 + _RBLK, r0:r0 + _RBLK].astype(f32)
            rr = jax.lax.broadcasted_iota(jnp.int32, (_RBLK, _RBLK), 0)
            cc = jax.lax.broadcasted_iota(jnp.int32, (_RBLK, _RBLK), 1)
            dfix = jnp.where(rr == cc, 1.0, dsub)
            A_ref[r0:r0 + _RBLK, r0:r0 + _RBLK] = dfix.astype(bf16)
            corrs.append(jnp.sum(dfix - dsub, axis=0, keepdims=True))
        dinv = jax.lax.rsqrt(deg_ref[...] + jnp.concatenate(corrs, axis=1))
        dv09 = (1.0 - _ALPHA) * dinv

        usa_ref[...] = (dinv * ht_ref[...]).astype(bf16)

        us_refs = [usa_ref, usb_ref]
        for l in range(_L):
            beta = math.log(_THETA / (l + 1) + 1.0)
            g = 1.0 - beta
            src = us_refs[l % 2]
            dst = us_refs[(l + 1) % 2]
            Wl = ((beta / g) * jnp.transpose(Wc_ref[l])).astype(bf16)
            dvg = g * dinv
            us = src[...]
            for j in range(_N // _CHUNK):
                c0 = j * _CHUNK
                mm = jnp.dot(us, A_ref[:, c0:c0 + _CHUNK],
                             preferred_element_type=f32)
                m = mm * dv09[:, c0:c0 + _CHUNK] + h0a_ref[:, c0:c0 + _CHUNK]
                mt_ref[:, c0:c0 + _CHUNK] = m
            for j in range(_N // _CHUNK):
                c0 = j * _CHUNK
                mc = mt_ref[:, c0:c0 + _CHUNK]
                s = jnp.dot(Wl, mc.astype(bf16),
                            preferred_element_type=f32)
                hs = jnp.maximum(mc + s, 0.0)
                if l < _L - 1:
                    dst[:, c0:c0 + _CHUNK] = (hs * dvg[:, c0:c0 + _CHUNK]
                                              ).astype(bf16)
                else:
                    ht_ref[:, c0:c0 + _CHUNK] = hs

        g_last = 1.0 - math.log(_THETA / _L + 1.0)
        W1g = (g_last * jnp.transpose(W1_ref[...])).astype(bf16)
        htf = jnp.transpose(ht_ref[...])
        out_ref[...] = jnp.dot(htf.astype(bf16), W1g,
                               preferred_element_type=f32) + b1_ref[...]


def _run(x, adj, W0, b0r, W1, b1r, Wc, interpret=False):
    return pl.pallas_call(
        _gcnii_body,
        grid=(_STEPS + 1,),
        in_specs=[
            pl.BlockSpec((_N, _F), lambda i: (0, 0)),
            pl.BlockSpec((_RBLK, _N),
                         lambda i: (jnp.minimum(i, _STEPS - 1), 0)),
            pl.BlockSpec((_F, _F), lambda i: (0, 0)),
            pl.BlockSpec((1, _F), lambda i: (0, 0)),
            pl.BlockSpec((_F, _F), lambda i: (0, 0)),
            pl.BlockSpec((1, _F), lambda i: (0, 0)),
            pl.BlockSpec((_L, _F, _F), lambda i: (0, 0, 0)),
        ],
        out_specs=pl.BlockSpec((_N, _F), lambda i: (0, 0)),
        out_shape=jax.ShapeDtypeStruct((_N, _F), jnp.float32),
        scratch_shapes=[
            pltpu.VMEM((_N, _N), jnp.bfloat16),
            pltpu.VMEM((_F, _N), jnp.bfloat16),
            pltpu.VMEM((_F, _N), jnp.bfloat16),
            pltpu.VMEM((_F, _N), jnp.float32),
            pltpu.VMEM((_F, _N), jnp.float32),
            pltpu.VMEM((_F, _N), jnp.float32),
            pltpu.VMEM((1, _N), jnp.float32),
        ],
        interpret=interpret,
    )(x, adj, W0, b0r, W1, b1r, Wc)


def kernel(x, adj, W0, b0, W1, b1, Wc):
    return _run(x, adj, W0, b0.reshape(1, _F), W1, b1.reshape(1, _F), Wc)


# speedup vs baseline: 1.0909x; 1.0482x over previous
import math

import jax
import jax.numpy as jnp
from jax.experimental import pallas as pl
from jax.experimental.pallas import tpu as pltpu

_N = 2048
_F = 256
_L = 8
_ALPHA = 0.1
_THETA = 0.5
_RBLK = 512
_CHUNK = 512
_STEPS = _N // _RBLK


def _gcnii_body(x_ref, adj_ref, W0_ref, b0_ref, W1_ref, b1_ref, Wc_ref,
                out_ref, A_ref, usa_ref, usb_ref, h0a_ref, ht_ref,
                mt_ref, deg_ref):
    f32 = jnp.float32
    bf16 = jnp.bfloat16
    step = pl.program_id(0)

    @pl.when(step < _STEPS)
    def _build():
        a = adj_ref[...].astype(f32)
        prev = jnp.where(step == 0, jnp.zeros((1, _N), f32), deg_ref[...])
        deg_ref[...] = prev + jnp.sum(a, axis=0, keepdims=True)
        A_ref[pl.ds(step * _RBLK, _RBLK), :] = a.astype(bf16)

    @pl.when(step == 0)
    def _input_layer():
        xt = jnp.transpose(x_ref[...])
        b0c = jnp.transpose(b0_ref[...])
        h0 = jnp.dot(W0_ref[...].astype(bf16), xt.astype(bf16),
                     preferred_element_type=f32) + b0c
        h0 = jnp.maximum(h0, 0.0)
        ht_ref[...] = h0
        h0a_ref[...] = _ALPHA * h0

    @pl.when(step == _STEPS)
    def _layers():
        corrs = []
        for i in range(_STEPS):
            r0 = i * _RBLK
            dsub = A_ref[r0:r0 + _RBLK, r0:r0 + _RBLK].astype(f32)
            rr = jax.lax.broadcasted_iota(jnp.int32, (_RBLK, _RBLK), 0)
            cc = jax.lax.broadcasted_iota(jnp.int32, (_RBLK, _RBLK), 1)
            dfix = jnp.where(rr == cc, 1.0, dsub)
            A_ref[r0:r0 + _RBLK, r0:r0 + _RBLK] = dfix.astype(bf16)
            corrs.append(jnp.sum(dfix - dsub, axis=0, keepdims=True))
        dinv = jax.lax.rsqrt(deg_ref[...] + jnp.concatenate(corrs, axis=1))
        dv09 = (1.0 - _ALPHA) * dinv

        usa_ref[...] = (dinv * ht_ref[...]).astype(bf16)

        us_refs = [usa_ref, usb_ref]
        for l in range(_L):
            beta = math.log(_THETA / (l + 1) + 1.0)
            g = 1.0 - beta
            src = us_refs[l % 2]
            dst = us_refs[(l + 1) % 2]
            Wl = ((beta / g) * jnp.transpose(Wc_ref[l])).astype(bf16)
            dvg = g * dinv
            us = src[...]
            for j in range(_N // _CHUNK):
                c0 = j * _CHUNK
                mm = jnp.dot(us, A_ref[:, c0:c0 + _CHUNK],
                             preferred_element_type=f32)
                m = mm * dv09[:, c0:c0 + _CHUNK] + h0a_ref[:, c0:c0 + _CHUNK]
                mt_ref[:, c0:c0 + _CHUNK] = m
            for j in range(_N // _CHUNK):
                c0 = j * _CHUNK
                mc = mt_ref[:, c0:c0 + _CHUNK]
                s = jnp.dot(Wl, mc.astype(bf16),
                            preferred_element_type=f32)
                hs = jnp.maximum(mc + s, 0.0)
                if l < _L - 1:
                    dst[:, c0:c0 + _CHUNK] = (hs * dvg[:, c0:c0 + _CHUNK]
                                              ).astype(bf16)
                else:
                    ht_ref[:, c0:c0 + _CHUNK] = hs

        g_last = 1.0 - math.log(_THETA / _L + 1.0)
        W1g = (g_last * jnp.transpose(W1_ref[...])).astype(bf16)
        htf = jnp.transpose(ht_ref[...])
        out_ref[...] = jnp.dot(htf.astype(bf16), W1g,
                               preferred_element_type=f32) + b1_ref[...]


def _run(x, adj, W0, b0r, W1, b1r, Wc, interpret=False):
    return pl.pallas_call(
        _gcnii_body,
        grid=(_STEPS + 1,),
        in_specs=[
            pl.BlockSpec((_N, _F), lambda i: (0, 0)),
            pl.BlockSpec((_RBLK, _N),
                         lambda i: (jnp.minimum(i, _STEPS - 1), 0)),
            pl.BlockSpec((_F, _F), lambda i: (0, 0)),
            pl.BlockSpec((1, _F), lambda i: (0, 0)),
            pl.BlockSpec((_F, _F), lambda i: (0, 0)),
            pl.BlockSpec((1, _F), lambda i: (0, 0)),
            pl.BlockSpec((_L, _F, _F), lambda i: (0, 0, 0)),
        ],
        out_specs=pl.BlockSpec((_N, _F), lambda i: (0, 0)),
        out_shape=jax.ShapeDtypeStruct((_N, _F), jnp.float32),
        scratch_shapes=[
            pltpu.VMEM((_N, _N), jnp.bfloat16),
            pltpu.VMEM((_F, _N), jnp.bfloat16),
            pltpu.VMEM((_F, _N), jnp.bfloat16),
            pltpu.VMEM((_F, _N), jnp.float32),
            pltpu.VMEM((_F, _N), jnp.float32),
            pltpu.VMEM((_F, _N), jnp.float32),
            pltpu.VMEM((1, _N), jnp.float32),
        ],
        interpret=interpret,
    )(x, adj, W0, b0r, W1, b1r, Wc)


def kernel(x, adj, W0, b0, W1, b1, Wc):
    return _run(x, adj, W0, b0.reshape(1, _F), W1, b1.reshape(1, _F), Wc)


# speedup vs baseline: 1.0967x; 1.0053x over previous
import math

import jax
import jax.numpy as jnp
from jax.experimental import pallas as pl
from jax.experimental.pallas import tpu as pltpu

_N = 2048
_F = 256
_L = 8
_ALPHA = 0.1
_THETA = 0.5
_RBLK = 1024
_CHUNK = 512
_STEPS = _N // _RBLK


def _gcnii_body(x_ref, adj_ref, W0_ref, b0_ref, W1_ref, b1_ref, Wc_ref,
                out_ref, A_ref, usa_ref, usb_ref, h0a_ref, ht_ref,
                mt_ref, deg_ref):
    f32 = jnp.float32
    bf16 = jnp.bfloat16
    step = pl.program_id(0)

    @pl.when(step < _STEPS)
    def _build():
        a = adj_ref[...].astype(f32)
        prev = jnp.where(step == 0, jnp.zeros((1, _N), f32), deg_ref[...])
        deg_ref[...] = prev + jnp.sum(a, axis=0, keepdims=True)
        A_ref[pl.ds(step * _RBLK, _RBLK), :] = a.astype(bf16)

    @pl.when(step == 0)
    def _input_layer():
        xt = jnp.transpose(x_ref[...])
        b0c = jnp.transpose(b0_ref[...])
        h0 = jnp.dot(W0_ref[...].astype(bf16), xt.astype(bf16),
                     preferred_element_type=f32) + b0c
        h0 = jnp.maximum(h0, 0.0)
        ht_ref[...] = h0
        h0a_ref[...] = _ALPHA * h0

    @pl.when(step == _STEPS)
    def _layers():
        corrs = []
        for i in range(_STEPS):
            r0 = i * _RBLK
            dsub = A_ref[r0:r0 + _RBLK, r0:r0 + _RBLK].astype(f32)
            rr = jax.lax.broadcasted_iota(jnp.int32, (_RBLK, _RBLK), 0)
            cc = jax.lax.broadcasted_iota(jnp.int32, (_RBLK, _RBLK), 1)
            dfix = jnp.where(rr == cc, 1.0, dsub)
            A_ref[r0:r0 + _RBLK, r0:r0 + _RBLK] = dfix.astype(bf16)
            corrs.append(jnp.sum(dfix - dsub, axis=0, keepdims=True))
        dinv = jax.lax.rsqrt(deg_ref[...] + jnp.concatenate(corrs, axis=1))
        dv09 = (1.0 - _ALPHA) * dinv

        usa_ref[...] = (dinv * ht_ref[...]).astype(bf16)

        us_refs = [usa_ref, usb_ref]
        for l in range(_L):
            beta = math.log(_THETA / (l + 1) + 1.0)
            g = 1.0 - beta
            src = us_refs[l % 2]
            dst = us_refs[(l + 1) % 2]
            Wl = ((beta / g) * jnp.transpose(Wc_ref[l])).astype(bf16)
            dvg = g * dinv
            us = src[...]
            for j in range(_N // _CHUNK):
                c0 = j * _CHUNK
                mm = jnp.dot(us, A_ref[:, c0:c0 + _CHUNK],
                             preferred_element_type=f32)
                m = mm * dv09[:, c0:c0 + _CHUNK] + h0a_ref[:, c0:c0 + _CHUNK]
                mt_ref[:, c0:c0 + _CHUNK] = m
            for j in range(_N // _CHUNK):
                c0 = j * _CHUNK
                mc = mt_ref[:, c0:c0 + _CHUNK]
                s = jnp.dot(Wl, mc.astype(bf16),
                            preferred_element_type=f32)
                hs = jnp.maximum(mc + s, 0.0)
                if l < _L - 1:
                    dst[:, c0:c0 + _CHUNK] = (hs * dvg[:, c0:c0 + _CHUNK]
                                              ).astype(bf16)
                else:
                    ht_ref[:, c0:c0 + _CHUNK] = hs

        g_last = 1.0 - math.log(_THETA / _L + 1.0)
        W1g = (g_last * jnp.transpose(W1_ref[...])).astype(bf16)
        htf = jnp.transpose(ht_ref[...])
        out_ref[...] = jnp.dot(htf.astype(bf16), W1g,
                               preferred_element_type=f32) + b1_ref[...]


def _run(x, adj, W0, b0r, W1, b1r, Wc, interpret=False):
    return pl.pallas_call(
        _gcnii_body,
        grid=(_STEPS + 1,),
        in_specs=[
            pl.BlockSpec((_N, _F), lambda i: (0, 0)),
            pl.BlockSpec((_RBLK, _N),
                         lambda i: (jnp.minimum(i, _STEPS - 1), 0)),
            pl.BlockSpec((_F, _F), lambda i: (0, 0)),
            pl.BlockSpec((1, _F), lambda i: (0, 0)),
            pl.BlockSpec((_F, _F), lambda i: (0, 0)),
            pl.BlockSpec((1, _F), lambda i: (0, 0)),
            pl.BlockSpec((_L, _F, _F), lambda i: (0, 0, 0)),
        ],
        out_specs=pl.BlockSpec((_N, _F), lambda i: (0, 0)),
        out_shape=jax.ShapeDtypeStruct((_N, _F), jnp.float32),
        scratch_shapes=[
            pltpu.VMEM((_N, _N), jnp.bfloat16),
            pltpu.VMEM((_F, _N), jnp.bfloat16),
            pltpu.VMEM((_F, _N), jnp.bfloat16),
            pltpu.VMEM((_F, _N), jnp.float32),
            pltpu.VMEM((_F, _N), jnp.float32),
            pltpu.VMEM((_F, _N), jnp.float32),
            pltpu.VMEM((1, _N), jnp.float32),
        ],
        interpret=interpret,
    )(x, adj, W0, b0r, W1, b1r, Wc)


def kernel(x, adj, W0, b0, W1, b1, Wc):
    return _run(x, adj, W0, b0.reshape(1, _F), W1, b1.reshape(1, _F), Wc)


# speedup vs baseline: 1.1243x; 1.0252x over previous
import math

import jax
import jax.numpy as jnp
from jax.experimental import pallas as pl
from jax.experimental.pallas import tpu as pltpu

_N = 2048
_F = 256
_L = 8
_ALPHA = 0.1
_THETA = 0.5
_RBLK = 1024
_CHUNK = 512
_STEPS = _N // _RBLK
_DBLK = 256


def _gcnii_body(x_ref, adj_ref, W0_ref, b0_ref, W1_ref, b1_ref, Wc_ref,
                out_ref, A_ref, usa_ref, usb_ref, h0a_ref, ht_ref,
                mt_ref, deg_ref):
    f32 = jnp.float32
    bf16 = jnp.bfloat16
    step = pl.program_id(0)

    @pl.when(step < _STEPS)
    def _build():
        a = adj_ref[...].astype(f32)
        prev = jnp.where(step == 0, jnp.zeros((1, _N), f32), deg_ref[...])
        deg_ref[...] = prev + jnp.sum(a, axis=0, keepdims=True)
        A_ref[pl.ds(step * _RBLK, _RBLK), :] = a.astype(bf16)

    @pl.when(step == 0)
    def _input_layer():
        xt = jnp.transpose(x_ref[...])
        b0c = jnp.transpose(b0_ref[...])
        h0 = jnp.dot(W0_ref[...].astype(bf16), xt.astype(bf16),
                     preferred_element_type=f32) + b0c
        h0 = jnp.maximum(h0, 0.0)
        ht_ref[...] = h0
        h0a_ref[...] = _ALPHA * h0

    @pl.when(step == _STEPS)
    def _layers():
        corrs = []
        for i in range(_N // _DBLK):
            r0 = i * _DBLK
            dsub = A_ref[r0:r0 + _DBLK, r0:r0 + _DBLK].astype(f32)
            rr = jax.lax.broadcasted_iota(jnp.int32, (_DBLK, _DBLK), 0)
            cc = jax.lax.broadcasted_iota(jnp.int32, (_DBLK, _DBLK), 1)
            dfix = jnp.where(rr == cc, 1.0, dsub)
            A_ref[r0:r0 + _DBLK, r0:r0 + _DBLK] = dfix.astype(bf16)
            corrs.append(jnp.sum(dfix - dsub, axis=0, keepdims=True))
        dinv = jax.lax.rsqrt(deg_ref[...] + jnp.concatenate(corrs, axis=1))
        dv09 = (1.0 - _ALPHA) * dinv

        usa_ref[...] = (dinv * ht_ref[...]).astype(bf16)

        us_refs = [usa_ref, usb_ref]
        for l in range(_L):
            beta = math.log(_THETA / (l + 1) + 1.0)
            g = 1.0 - beta
            src = us_refs[l % 2]
            dst = us_refs[(l + 1) % 2]
            Wl = ((beta / g) * jnp.transpose(Wc_ref[l])).astype(bf16)
            dvg = g * dinv
            us = src[...]
            for j in range(_N // _CHUNK):
                c0 = j * _CHUNK
                mm = jnp.dot(us, A_ref[:, c0:c0 + _CHUNK],
                             preferred_element_type=f32)
                m = mm * dv09[:, c0:c0 + _CHUNK] + h0a_ref[:, c0:c0 + _CHUNK]
                mt_ref[:, c0:c0 + _CHUNK] = m
            for j in range(_N // _CHUNK):
                c0 = j * _CHUNK
                mc = mt_ref[:, c0:c0 + _CHUNK]
                s = jnp.dot(Wl, mc.astype(bf16),
                            preferred_element_type=f32)
                hs = jnp.maximum(mc + s, 0.0)
                if l < _L - 1:
                    dst[:, c0:c0 + _CHUNK] = (hs * dvg[:, c0:c0 + _CHUNK]
                                              ).astype(bf16)
                else:
                    ht_ref[:, c0:c0 + _CHUNK] = hs

        g_last = 1.0 - math.log(_THETA / _L + 1.0)
        W1g = (g_last * jnp.transpose(W1_ref[...])).astype(bf16)
        htf = jnp.transpose(ht_ref[...])
        out_ref[...] = jnp.dot(htf.astype(bf16), W1g,
                               preferred_element_type=f32) + b1_ref[...]


def _run(x, adj, W0, b0r, W1, b1r, Wc, interpret=False):
    return pl.pallas_call(
        _gcnii_body,
        grid=(_STEPS + 1,),
        in_specs=[
            pl.BlockSpec((_N, _F), lambda i: (0, 0)),
            pl.BlockSpec((_RBLK, _N),
                         lambda i: (jnp.minimum(i, _STEPS - 1), 0)),
            pl.BlockSpec((_F, _F), lambda i: (0, 0)),
            pl.BlockSpec((1, _F), lambda i: (0, 0)),
            pl.BlockSpec((_F, _F), lambda i: (0, 0)),
            pl.BlockSpec((1, _F), lambda i: (0, 0)),
            pl.BlockSpec((_L, _F, _F), lambda i: (0, 0, 0)),
        ],
        out_specs=pl.BlockSpec((_N, _F), lambda i: (0, 0)),
        out_shape=jax.ShapeDtypeStruct((_N, _F), jnp.float32),
        scratch_shapes=[
            pltpu.VMEM((_N, _N), jnp.bfloat16),
            pltpu.VMEM((_F, _N), jnp.bfloat16),
            pltpu.VMEM((_F, _N), jnp.bfloat16),
            pltpu.VMEM((_F, _N), jnp.float32),
            pltpu.VMEM((_F, _N), jnp.float32),
            pltpu.VMEM((_F, _N), jnp.float32),
            pltpu.VMEM((1, _N), jnp.float32),
        ],
        interpret=interpret,
    )(x, adj, W0, b0r, W1, b1r, Wc)


def kernel(x, adj, W0, b0, W1, b1, Wc):
    return _run(x, adj, W0, b0.reshape(1, _F), W1, b1.reshape(1, _F), Wc)
